# baseline (speedup 1.0000x reference)
import functools

import jax
import jax.numpy as jnp
from jax import lax
from jax.experimental import pallas as pl
from jax.experimental.pallas import tpu as pltpu
from jax.experimental.pallas import tpu_sc as plsc

R, C = 128, 32768
L = 16
NC, NS = 2, 16
NW = NC * NS
SLAB = 8
HALF = C // 2
CHUNK = 2048
NCH = HALF // CHUNK


def _body(x_hbm, out_hbm, dump_hbm, ex_hbm, cbuf0, cbuf1, zbuf, pbuf, stg,
          pstg, sem_in, sem_z, sem_p):
    c = lax.axis_index("c")
    s = lax.axis_index("s")
    wid = c * NS + s
    slab = c * (NS // 2) + s // 2
    h = s % 2
    rr0 = pl.multiple_of(slab * SLAB, SLAB)
    cc0 = pl.multiple_of(h * HALF, 128)
    lanes = lax.iota(jnp.int32, L)
    zeros_v = jnp.zeros((L,), jnp.float32)
    ones_v = jnp.ones((L,), jnp.float32)
    lane0 = lanes == 0
    cbufs = [cbuf0, cbuf1]

    def chunk_src(ch):
        col = pl.multiple_of(cc0 + ch * CHUNK, 128)
        return x_hbm.at[pl.ds(rr0, SLAB), pl.ds(col, CHUNK)]

    def chunk_dst(ch):
        col = pl.multiple_of(cc0 + ch * CHUNK, 128)
        return out_hbm.at[pl.ds(rr0, SLAB), pl.ds(col, CHUNK)]

    cp_in = pltpu.async_copy(chunk_src(0), cbufs[0], sem_in)

    def zbody(i, _):
        for r in range(SLAB):
            zbuf[r, pl.ds(i * L, L)] = zeros_v
        return 0

    lax.fori_loop(0, CHUNK // L, zbody, 0)
    for r in range(SLAB):
        pbuf[pl.ds(r * L, L)] = zeros_v

    zcps = [
        pltpu.async_copy(zbuf, chunk_dst(ch), sem_z) for ch in range(NCH)
    ]

    neg_inf = jnp.full((L,), -jnp.inf, jnp.float32)
    vmaxs = [neg_inf] * SLAB
    vidxs = [jnp.zeros((L,), jnp.int32)] * SLAB
    for ch in range(NCH):
        cp_in.wait()
        if ch + 1 < NCH:
            cp_in = pltpu.async_copy(
                chunk_src(ch + 1), cbufs[(ch + 1) % 2], sem_in
            )
        buf = cbufs[ch % 2]
        cidx0 = (cc0 + ch * CHUNK) + lanes

        def sbody(i, carry):
            vm, vi, cidx = carry
            nvm, nvi = [], []
            for r in range(SLAB):
                v = buf[r, pl.ds(i * L, L)]
                m = v > vm[r]
                nvm.append(jnp.where(m, v, vm[r]))
                nvi.append(jnp.where(m, cidx, vi[r]))
            return tuple(nvm), tuple(nvi), cidx + L

        vm, vi, _ = lax.fori_loop(
            0, CHUNK // L, sbody, (tuple(vmaxs), tuple(vidxs), cidx0)
        )
        vmaxs, vidxs = list(vm), list(vi)

    mm = jnp.full((L,), -jnp.inf, jnp.float32)
    mi = jnp.zeros((L,), jnp.int32)
    for r in range(SLAB):
        gmax = jnp.max(vmaxs[r])
        idx = jnp.min(jnp.where(vmaxs[r] == gmax, vidxs[r], jnp.int32(C)))
        rv = jnp.full((L,), r, jnp.int32)
        mm = jnp.where(lanes == r, gmax, mm)
        mi = jnp.where(lanes == r, idx, mi)
        plsc.store_scatter(stg, [rv], jnp.full((L,), 0.0, jnp.float32) + gmax,
                           mask=lane0)
        plsc.store_scatter(
            stg, [rv + SLAB],
            plsc.bitcast(jnp.zeros((L,), jnp.int32) + idx, jnp.float32),
            mask=lane0)

    pltpu.sync_copy(stg, ex_hbm.at[wid, pl.ds(0, L)])
    plsc.subcore_barrier()
    pltpu.sync_copy(ex_hbm.at[wid ^ 1, pl.ds(0, L)], pstg)

    pall = pstg[...]
    pm = pall
    pi = plsc.bitcast(
        plsc.load_gather(pstg, [SLAB + (lanes & (SLAB - 1))]), jnp.int32
    )
    i_win = (mm > pm) | ((mm == pm) & (mi < pi))

    segs, owns = [], []
    for r in range(SLAB):
        rmask = lanes == r
        own = jnp.max(jnp.where(rmask & i_win, jnp.int32(1), 0)) > 0
        idx = jnp.min(jnp.where(rmask, mi, jnp.int32(C)))
        seg = pl.multiple_of((idx // L) * L, L)
        off = jnp.full((L,), r * L, jnp.int32) + (idx - seg)
        plsc.store_scatter(pbuf, [off], ones_v, mask=lane0)
        segs.append(seg)
        owns.append(own)

    for z in zcps:
        z.wait()
    for r in range(SLAB):
        src = pbuf.at[pl.ds(r * L, L)]

        @pl.when(owns[r])
        def _():
            pltpu.async_copy(
                src, out_hbm.at[rr0 + r, pl.ds(segs[r], L)], sem_p
            )

        @pl.when(jnp.logical_not(owns[r]))
        def _():
            pltpu.async_copy(src, dump_hbm.at[wid, pl.ds(0, L)], sem_p)

    for r in range(SLAB):
        pltpu.make_async_copy(
            x_hbm.at[0, pl.ds(0, L)], pbuf.at[pl.ds(r * L, L)], sem_p
        ).wait()


@jax.jit
def kernel(x):
    mesh = plsc.VectorSubcoreMesh(
        core_axis_name="c", subcore_axis_name="s", num_cores=NC, num_subcores=NS
    )
    f = functools.partial(
        pl.kernel,
        mesh=mesh,
        out_type=(
            jax.ShapeDtypeStruct((R, C), jnp.float32),
            jax.ShapeDtypeStruct((NW, 128), jnp.float32),
            jax.ShapeDtypeStruct((NW, 128), jnp.float32),
        ),
        scratch_types=[
            pltpu.VMEM((SLAB, CHUNK), jnp.float32),
            pltpu.VMEM((SLAB, CHUNK), jnp.float32),
            pltpu.VMEM((SLAB, CHUNK), jnp.float32),
            pltpu.VMEM((SLAB * L,), jnp.float32),
            pltpu.VMEM((L,), jnp.float32),
            pltpu.VMEM((L,), jnp.float32),
            pltpu.SemaphoreType.DMA,
            pltpu.SemaphoreType.DMA,
            pltpu.SemaphoreType.DMA,
        ],
        compiler_params=pltpu.CompilerParams(needs_layout_passes=False),
    )(_body)
    return f(x)[0]
